# Initial kernel scaffold; baseline (speedup 1.0000x reference)
import jax
import jax.numpy as jnp
from jax.experimental import pallas as pl


def kernel(x, edge_index, edge_label_index, p, W_ih, W_hh, b_ih, b_hh, W0, post_W, post_b):
    raise NotImplementedError("write your pallas kernel here")


# speedup vs baseline: 10.6552x; 10.6552x over previous
import functools

import jax
import jax.numpy as jnp
from jax import lax
from jax.experimental import pallas as pl
from jax.experimental.pallas import tpu as pltpu
from jax.experimental.pallas import tpu_sc as plsc

N = 10000
D = 128
E = 320000
L_LAB = 200000

NC = 2
NS = 16
NW = NC * NS
LANE = 16
CHUNK = 128
EC = E // CHUNK
LC = 1564
ROWS_PER_TILE = N // NS


@functools.lru_cache(maxsize=1)
def _sc_mesh():
    return plsc.VectorSubcoreMesh(
        core_axis_name="c", subcore_axis_name="s",
        num_cores=NC, num_subcores=NS)


def _wid():
    return lax.axis_index("s") * NC + lax.axis_index("c")


def _deg_body(col_hbm, ones_hbm, zeros_hbm, deg_out, cidx_v, ones_v, deg_sh):
    cid = lax.axis_index("c")
    sid = lax.axis_index("s")
    wid = _wid()
    r0 = sid * ROWS_PER_TILE
    pltpu.sync_copy(ones_hbm, ones_v)
    pltpu.sync_copy(zeros_hbm.at[pl.ds(r0, ROWS_PER_TILE)],
                    deg_sh.at[pl.ds(r0, ROWS_PER_TILE)])
    plsc.subcore_barrier()

    def body(k, carry):
        ci = wid + NW * k
        pltpu.sync_copy(col_hbm.at[ci], cidx_v)
        pltpu.sync_copy(ones_v, deg_sh.at[cidx_v], add=True)
        return carry

    n_w = jnp.where(wid < EC - (EC // NW) * NW, EC // NW + 1, EC // NW)
    lax.fori_loop(0, n_w, body, 0)
    plsc.subcore_barrier()
    pltpu.sync_copy(deg_sh.at[pl.ds(r0, ROWS_PER_TILE)],
                    deg_out.at[cid, pl.ds(r0, ROWS_PER_TILE)])


def _deg_call(col2d, ones_rows, zeros_n16):
    f = functools.partial(
        pl.kernel,
        out_type=jax.ShapeDtypeStruct((NC, N, LANE), jnp.float32),
        mesh=_sc_mesh(),
        compiler_params=pltpu.CompilerParams(
            needs_layout_passes=False, use_tc_tiling_on_sc=False),
        scratch_types=[
            pltpu.VMEM((CHUNK,), jnp.int32),
            pltpu.VMEM((CHUNK, LANE), jnp.float32),
            pltpu.VMEM_SHARED((N, LANE), jnp.float32),
        ],
    )(_deg_body)
    return f(col2d, ones_rows, zeros_n16)


def _evolve_body(x_ref, p_ref, wih_ref, whh_ref, bih_ref, bhh_ref,
                 w0_ref, d0_ref, d1_ref, y_ref, dinv_ref, oh_ref):
    p = p_ref[...]
    sg0 = jnp.dot(x_ref[...], p,
                  preferred_element_type=jnp.float32).reshape(625, 16)
    flat = (lax.broadcasted_iota(jnp.int32, (625, 16), 0) * 16
            + lax.broadcasted_iota(jnp.int32, (625, 16), 1))
    lane128 = lax.broadcasted_iota(jnp.int32, (1, D), 1)
    flat_n = lax.broadcasted_iota(jnp.int32, (1, N), 1)
    NEG = jnp.float32(-3.0e38)
    BIG = jnp.int32(2 ** 30)

    def step(i, carry):
        sg, topv = carry
        m = jnp.max(sg)
        idx = jnp.min(jnp.where(sg == m, flat, BIG))
        topv = jnp.where(lane128 == i, m, topv)
        row = (flat_n == idx).astype(jnp.float32)
        oh_ref[pl.ds(i, 1), :] = row
        sg = jnp.where(flat == idx, NEG, sg)
        return sg, topv

    _, topv = lax.fori_loop(
        0, D, step, (sg0, jnp.zeros((1, D), jnp.float32)))

    inv_norm = lax.rsqrt(jnp.sum(p * p))
    tv = jnp.tanh(topv * inv_norm)
    x = x_ref[...]
    xperm = jnp.dot(oh_ref[...], x, preferred_element_type=jnp.float32,
                    precision=lax.Precision.HIGHEST)
    eye = (lax.broadcasted_iota(jnp.int32, (D, D), 0)
           == lax.broadcasted_iota(jnp.int32, (D, D), 1)).astype(jnp.float32)
    diagm = eye * tv
    x_tilde = jnp.dot(diagm, xperm, preferred_element_type=jnp.float32,
                      precision=lax.Precision.HIGHEST)

    w0 = w0_ref[...]
    gi = lax.dot_general(x_tilde, wih_ref[...], (((1,), (1,)), ((), ())),
                         preferred_element_type=jnp.float32,
                         precision=lax.Precision.HIGHEST) + bih_ref[...]
    gh = lax.dot_general(w0, whh_ref[...], (((1,), (1,)), ((), ())),
                         preferred_element_type=jnp.float32,
                         precision=lax.Precision.HIGHEST) + bhh_ref[...]
    i_r, i_z, i_n = gi[:, :D], gi[:, D:2 * D], gi[:, 2 * D:]
    h_r, h_z, h_n = gh[:, :D], gh[:, D:2 * D], gh[:, 2 * D:]
    r = jax.nn.sigmoid(i_r + h_r)
    z = jax.nn.sigmoid(i_z + h_z)
    n = jnp.tanh(i_n + r * h_n)
    W = (1.0 - z) * n + z * w0

    xw = jnp.dot(x, W, preferred_element_type=jnp.float32)
    degs = d0_ref[...] + d1_ref[...]
    dinv = jnp.where(degs > 0, lax.rsqrt(jnp.maximum(degs, 1e-12)), 0.0)
    dinv_ref[...] = dinv
    y_ref[...] = xw * dinv


def _evolve_call(x, p2, W_ih, W_hh, bih2, bhh2, W0, d0, d1):
    return pl.pallas_call(
        _evolve_body,
        out_shape=[
            jax.ShapeDtypeStruct((N, D), jnp.float32),
            jax.ShapeDtypeStruct((N, 1), jnp.float32),
        ],
        scratch_shapes=[pltpu.VMEM((D, N), jnp.float32)],
    )(x, p2, W_ih, W_hh, bih2, bhh2, W0, d0, d1)


def _scatter_body(row_hbm, col_hbm, y_hbm, zeros_hbm, acc_out,
                  ridx_v, cidx_v, rows_v, acc_sh, sem):
    cid = lax.axis_index("c")
    sid = lax.axis_index("s")
    wid = _wid()
    r0 = sid * ROWS_PER_TILE
    pltpu.sync_copy(zeros_hbm.at[pl.ds(r0, ROWS_PER_TILE)],
                    acc_sh.at[pl.ds(r0, ROWS_PER_TILE)])
    plsc.subcore_barrier()

    def body(k, carry):
        ci = wid + NW * k
        pltpu.sync_copy(row_hbm.at[ci], ridx_v)
        pltpu.sync_copy(col_hbm.at[ci], cidx_v)
        pltpu.async_copy(y_hbm.at[ridx_v], rows_v, sem).wait()
        pltpu.sync_copy(rows_v, acc_sh.at[cidx_v], add=True)
        return carry

    n_w = jnp.where(wid < EC - (EC // NW) * NW, EC // NW + 1, EC // NW)
    lax.fori_loop(0, n_w, body, 0)
    plsc.subcore_barrier()
    pltpu.sync_copy(acc_sh.at[pl.ds(r0, ROWS_PER_TILE)],
                    acc_out.at[cid, pl.ds(r0, ROWS_PER_TILE)])


def _scatter_call(row2d, col2d, y, zeros_nd):
    f = functools.partial(
        pl.kernel,
        out_type=jax.ShapeDtypeStruct((NC, N, D), jnp.float32),
        mesh=_sc_mesh(),
        compiler_params=pltpu.CompilerParams(
            needs_layout_passes=False, use_tc_tiling_on_sc=False),
        scratch_types=[
            pltpu.VMEM((CHUNK,), jnp.int32),
            pltpu.VMEM((CHUNK,), jnp.int32),
            pltpu.VMEM((CHUNK, D), jnp.float32),
            pltpu.VMEM_SHARED((N, D), jnp.float32),
            pltpu.SemaphoreType.DMA,
        ],
    )(_scatter_body)
    return f(row2d, col2d, y, zeros_nd)


def _post_body(acc0_ref, acc1_ref, dinv_ref, pw_ref, h_ref, hw_ref):
    a = acc0_ref[...] + acc1_ref[...]
    h = jnp.maximum(a * dinv_ref[...], 0.0)
    wsum = pw_ref[0:1, :] + pw_ref[1:2, :]
    h_ref[...] = h
    hw_ref[...] = h * wsum


def _post_call(acc0, acc1, dinv, post_W):
    return pl.pallas_call(
        _post_body,
        out_shape=[
            jax.ShapeDtypeStruct((N, D), jnp.float32),
            jax.ShapeDtypeStruct((N, D), jnp.float32),
        ],
    )(acc0, acc1, dinv, post_W)


def _score_body(sidx_hbm, didx_hbm, hw_hbm, h_hbm, out_hbm,
                si_v, di_v, bufS, bufD, outv, semS, semD):
    wid = _wid()
    nfull = LC // NW
    n_w = jnp.where(wid < LC - nfull * NW, nfull + 1, nfull)

    def chunk(k, carry):
        ci = wid + NW * k
        pltpu.sync_copy(sidx_hbm.at[ci], si_v)
        pltpu.sync_copy(didx_hbm.at[ci], di_v)
        cS = pltpu.async_copy(hw_hbm.at[si_v], bufS, semS)
        cD = pltpu.async_copy(h_hbm.at[di_v], bufD, semD)
        cS.wait()
        cD.wait()

        def grp(g, carry2):
            for e in range(LANE):
                row = g * LANE + e
                acc = jnp.zeros((LANE,), jnp.float32)
                for c in range(D // LANE):
                    s = bufS[row, pl.ds(c * LANE, LANE)]
                    d = bufD[row, pl.ds(c * LANE, LANE)]
                    acc = acc + s * d
                outv[pl.ds(g * (LANE * LANE) + e * LANE, LANE)] = acc
            return carry2

        lax.fori_loop(0, CHUNK // LANE, grp, 0)
        pltpu.sync_copy(outv, out_hbm.at[ci])
        return carry

    lax.fori_loop(0, n_w, chunk, 0)


def _score_call(sidx2d, didx2d, hw, h):
    f = functools.partial(
        pl.kernel,
        out_type=jax.ShapeDtypeStruct((LC, CHUNK * LANE), jnp.float32),
        mesh=_sc_mesh(),
        compiler_params=pltpu.CompilerParams(
            needs_layout_passes=False, use_tc_tiling_on_sc=False),
        scratch_types=[
            pltpu.VMEM((CHUNK,), jnp.int32),
            pltpu.VMEM((CHUNK,), jnp.int32),
            pltpu.VMEM((CHUNK, D), jnp.float32),
            pltpu.VMEM((CHUNK, D), jnp.float32),
            pltpu.VMEM((CHUNK * LANE,), jnp.float32),
            pltpu.SemaphoreType.DMA,
            pltpu.SemaphoreType.DMA,
        ],
    )(_score_body)
    return f(sidx2d, didx2d, hw, h)


_FR = LC * CHUNK * LANE // 256


def _reduce_body(rr_ref, pb_ref, out_ref):
    blockdiag = (lax.broadcasted_iota(jnp.int32, (256, LANE), 0) // LANE
                 == lax.broadcasted_iota(jnp.int32, (256, LANE), 1)
                 ).astype(jnp.float32)
    bsum = jnp.sum(pb_ref[...])
    out_ref[...] = jnp.dot(rr_ref[...], blockdiag,
                           preferred_element_type=jnp.float32) + bsum


def _reduce_call(rr, post_b2):
    return pl.pallas_call(
        _reduce_body,
        out_shape=jax.ShapeDtypeStruct((_FR, LANE), jnp.float32),
    )(rr, post_b2)


def kernel(x, edge_index, edge_label_index, p, W_ih, W_hh, b_ih, b_hh, W0,
           post_W, post_b):
    ei = edge_index.astype(jnp.int32)
    row2d = ei[0].reshape(EC, CHUNK)
    col2d = ei[1].reshape(EC, CHUNK)
    eli = edge_label_index.astype(jnp.int32)
    pad = LC * CHUNK - L_LAB
    sidx2d = jnp.concatenate(
        [eli[0], jnp.zeros((pad,), jnp.int32)]).reshape(LC, CHUNK)
    didx2d = jnp.concatenate(
        [eli[1], jnp.zeros((pad,), jnp.int32)]).reshape(LC, CHUNK)

    ones_rows = jnp.ones((CHUNK, LANE), jnp.float32)
    zeros_n16 = jnp.zeros((N, LANE), jnp.float32)
    zeros_nd = jnp.zeros((N, D), jnp.float32)

    deg = _deg_call(col2d, ones_rows, zeros_n16)
    d0 = deg[0, :, 0:1]
    d1 = deg[1, :, 0:1]

    p2 = p.reshape(D, 1)
    bih2 = b_ih.reshape(1, 3 * D)
    bhh2 = b_hh.reshape(1, 3 * D)
    y, dinv = _evolve_call(x, p2, W_ih, W_hh, bih2, bhh2, W0, d0, d1)

    acc = _scatter_call(row2d, col2d, y, zeros_nd)
    h, hw = _post_call(acc[0], acc[1], dinv, post_W)

    parts = _score_call(sidx2d, didx2d, hw, h)
    rr = parts.reshape(_FR, 256)
    out16 = _reduce_call(rr, post_b.reshape(1, 2))
    return out16.reshape(-1)[:L_LAB]
